# baseline (device time: 97142 ns/iter reference)
import jax
import jax.numpy as jnp
from jax import lax
from jax.experimental import pallas as pl
from jax.experimental.pallas import tpu as pltpu

N_DEV = 4
B, Sq, Skv, Dh = 4, 256, 1024, 128
H = 8
HG = 32
D = 1024
SCALE = 0.08838834764831843
ROWS = B * Sq
CHUNK = ROWS // N_DEV


def kernel(x, Wq, Wo, K_ext, V_ext):
    xm = x.reshape(ROWS, D)

    def body(x_ref, wq_ref, wo_ref, k_hbm, v_hbm, out_ref,
             kbuf, vbuf, comm_ref, q_ref, attn_ref,
             copy_sems, rs_send_sems, rs_recv_sems, bc_send_sems,
             bc_recv_sems):
        my = lax.axis_index("i")
        left = (my + N_DEV - 1) % N_DEV
        right = (my + 1) % N_DEV
        diag = (my + 2) % N_DEV
        hs = my * H

        barrier_sem = pltpu.get_barrier_semaphore()
        for nbr in (left, right):
            pl.semaphore_signal(
                barrier_sem, inc=1,
                device_id=(nbr,), device_id_type=pl.DeviceIdType.MESH,
            )
        pl.semaphore_wait(barrier_sem, 2)

        def kv_copy(b, slot):
            ck = pltpu.make_async_copy(
                k_hbm.at[b, :, pl.ds(hs, H), :], kbuf.at[slot],
                copy_sems.at[2 * slot])
            cv = pltpu.make_async_copy(
                v_hbm.at[b, :, pl.ds(hs, H), :], vbuf.at[slot],
                copy_sems.at[2 * slot + 1])
            ck.start()
            cv.start()
            return ck, cv

        def batch_index(j):
            if j < N_DEV - 1:
                return (my + N_DEV - j) % N_DEV
            return (my + 1) % N_DEV

        bf16 = jnp.bfloat16
        wqb = wq_ref[...].astype(bf16)
        wob = wo_ref[...].astype(bf16)

        rdmas = []
        pending = kv_copy(batch_index(0), 0)
        for j in range(N_DEV):
            b = batch_index(j)
            slot = j % 2
            q_ref[...] = jnp.dot(
                x_ref[pl.ds(b * Sq, Sq), :].astype(bf16), wqb,
                preferred_element_type=jnp.float32)
            pending[0].wait()
            pending[1].wait()
            if j + 1 < N_DEV:
                pending = kv_copy(batch_index(j + 1), (j + 1) % 2)

            def head_step(h, _, slot=slot):
                qh = q_ref[:, pl.ds(h * Dh, Dh)].astype(bf16)
                kh = kbuf[slot, :, h, :].astype(bf16)
                vh = vbuf[slot, :, h, :].astype(bf16)
                s = lax.dot_general(
                    qh, kh, (((1,), (1,)), ((), ())),
                    preferred_element_type=jnp.float32) * SCALE
                p = jnp.exp(s)
                l = jnp.sum(p, axis=-1, keepdims=True)
                o = jnp.dot(p.astype(bf16), vh,
                            preferred_element_type=jnp.float32) / l
                attn_ref[:, pl.ds(h * Dh, Dh)] = o.astype(bf16)
                return 0

            lax.fori_loop(0, H, head_step, 0)
            part = jnp.dot(attn_ref[...], wob,
                           preferred_element_type=jnp.float32)

            if j == 0:
                out_ref[pl.ds(b * CHUNK, CHUNK), :] = part
            else:
                prev = rdmas[j - 1]
                prev.wait_recv()
                out_ref[pl.ds(b * CHUNK, CHUNK), :] = (
                    part + comm_ref[j - 1])
            if j < N_DEV - 1:
                rdma = pltpu.make_async_remote_copy(
                    src_ref=out_ref.at[pl.ds(b * CHUNK, CHUNK)],
                    dst_ref=comm_ref.at[j],
                    send_sem=rs_send_sems.at[j],
                    recv_sem=rs_recv_sems.at[j],
                    device_id=(right,),
                    device_id_type=pl.DeviceIdType.MESH,
                )
                rdma.start()
                rdmas.append(rdma)

        own = (my + 1) % N_DEV
        own_slice = pl.ds(own * CHUNK, CHUNK)
        for k, tgt in enumerate((right, left, diag)):
            bc = pltpu.make_async_remote_copy(
                src_ref=out_ref.at[own_slice],
                dst_ref=out_ref.at[own_slice],
                send_sem=bc_send_sems.at[k],
                recv_sem=bc_recv_sems.at[k],
                device_id=(tgt,),
                device_id_type=pl.DeviceIdType.MESH,
            )
            bc.start()
            rdmas.append(bc)
        for k in range(3):
            rdmas[N_DEV - 1 + k].wait_recv()
        for r in rdmas:
            r.wait_send()

    out2 = pl.pallas_call(
        body,
        out_shape=jax.ShapeDtypeStruct((ROWS, D), jnp.float32),
        in_specs=[
            pl.BlockSpec(memory_space=pltpu.VMEM),
            pl.BlockSpec(memory_space=pltpu.VMEM),
            pl.BlockSpec(memory_space=pltpu.VMEM),
            pl.BlockSpec(memory_space=pl.ANY),
            pl.BlockSpec(memory_space=pl.ANY),
        ],
        out_specs=pl.BlockSpec(memory_space=pltpu.VMEM),
        scratch_shapes=[
            pltpu.VMEM((2, Skv, H, Dh), jnp.float32),
            pltpu.VMEM((2, Skv, H, Dh), jnp.float32),
            pltpu.VMEM((N_DEV - 1, CHUNK, D), jnp.float32),
            pltpu.VMEM((Sq, H * Dh), jnp.float32),
            pltpu.VMEM((Sq, H * Dh), jnp.bfloat16),
            pltpu.SemaphoreType.DMA((4,)),
            pltpu.SemaphoreType.DMA((N_DEV - 1,)),
            pltpu.SemaphoreType.DMA((N_DEV - 1,)),
            pltpu.SemaphoreType.DMA((3,)),
            pltpu.SemaphoreType.DMA((3,)),
        ],
        compiler_params=pltpu.CompilerParams(collective_id=0),
    )(xm, Wq, Wo, K_ext, V_ext)
    return out2.reshape(B, Sq, D)


# device time: 85904 ns/iter; 1.1308x vs baseline; 1.1308x over previous
import jax
import jax.numpy as jnp
from jax import lax
from jax.experimental import pallas as pl
from jax.experimental.pallas import tpu as pltpu

N_DEV = 4
B, Sq, Skv, Dh = 4, 256, 1024, 128
H = 8
HG = 32
D = 1024
SCALE = 0.08838834764831843
ROWS = B * Sq
CHUNK = ROWS // N_DEV


def kernel(x, Wq, Wo, K_ext, V_ext):
    xm = x.reshape(ROWS, D)

    def body(x_ref, wq_ref, wo_ref, k_hbm, v_hbm, out_ref,
             kbuf, vbuf, comm_ref, q_ref, attn_ref,
             copy_sems, rs_send_sems, rs_recv_sems, bc_send_sems,
             bc_recv_sems):
        my = lax.axis_index("i")
        left = (my + N_DEV - 1) % N_DEV
        right = (my + 1) % N_DEV
        diag = (my + 2) % N_DEV
        hs = my * H

        barrier_sem = pltpu.get_barrier_semaphore()
        for nbr in (left, right):
            pl.semaphore_signal(
                barrier_sem, inc=1,
                device_id=(nbr,), device_id_type=pl.DeviceIdType.MESH,
            )
        pl.semaphore_wait(barrier_sem, 2)

        def kv_copy(b, slot):
            copies = []
            for h in range(H):
                copies.append(pltpu.make_async_copy(
                    k_hbm.at[b, :, hs + h, :], kbuf.at[slot, h],
                    copy_sems.at[2 * slot]))
                copies.append(pltpu.make_async_copy(
                    v_hbm.at[b, :, hs + h, :], vbuf.at[slot, h],
                    copy_sems.at[2 * slot + 1]))
            for c in copies:
                c.start()
            return copies

        def batch_index(j):
            if j < N_DEV - 1:
                return (my + N_DEV - j) % N_DEV
            return (my + 1) % N_DEV

        bf16 = jnp.bfloat16
        wqb = wq_ref[...].astype(bf16)
        wob = wo_ref[...].astype(bf16)

        rdmas = []
        pending = kv_copy(batch_index(0), 0)
        for j in range(N_DEV):
            b = batch_index(j)
            slot = j % 2
            q_ref[...] = (jnp.dot(
                x_ref[pl.ds(b * Sq, Sq), :].astype(bf16), wqb,
                preferred_element_type=jnp.float32)
                * SCALE).astype(bf16)
            for c in pending:
                c.wait()
            if j + 1 < N_DEV:
                pending = kv_copy(batch_index(j + 1), (j + 1) % 2)

            def head_step(h, _, slot=slot):
                qh = q_ref[:, pl.ds(h * Dh, Dh)]
                kh = kbuf[slot, h].astype(bf16)
                vh = vbuf[slot, h].astype(bf16)
                s = lax.dot_general(
                    qh, kh, (((1,), (1,)), ((), ())),
                    preferred_element_type=jnp.float32)
                p = jnp.exp(s)
                l = jnp.sum(p, axis=-1, keepdims=True)
                o = jnp.dot(p.astype(bf16), vh,
                            preferred_element_type=jnp.float32) / l
                attn_ref[:, pl.ds(h * Dh, Dh)] = o.astype(bf16)
                return 0

            lax.fori_loop(0, H, head_step, 0, unroll=4)
            part = jnp.dot(attn_ref[...], wob,
                           preferred_element_type=jnp.float32)

            if j == 0:
                out_ref[pl.ds(b * CHUNK, CHUNK), :] = part
            else:
                prev = rdmas[j - 1]
                prev.wait_recv()
                out_ref[pl.ds(b * CHUNK, CHUNK), :] = (
                    part + comm_ref[j - 1])
            if j < N_DEV - 1:
                rdma = pltpu.make_async_remote_copy(
                    src_ref=out_ref.at[pl.ds(b * CHUNK, CHUNK)],
                    dst_ref=comm_ref.at[j],
                    send_sem=rs_send_sems.at[j],
                    recv_sem=rs_recv_sems.at[j],
                    device_id=(right,),
                    device_id_type=pl.DeviceIdType.MESH,
                )
                rdma.start()
                rdmas.append(rdma)

        own = (my + 1) % N_DEV
        own_slice = pl.ds(own * CHUNK, CHUNK)
        for k, tgt in enumerate((right, left, diag)):
            bc = pltpu.make_async_remote_copy(
                src_ref=out_ref.at[own_slice],
                dst_ref=out_ref.at[own_slice],
                send_sem=bc_send_sems.at[k],
                recv_sem=bc_recv_sems.at[k],
                device_id=(tgt,),
                device_id_type=pl.DeviceIdType.MESH,
            )
            bc.start()
            rdmas.append(bc)
        for k in range(3):
            rdmas[N_DEV - 1 + k].wait_recv()
        for r in rdmas:
            r.wait_send()

    out2 = pl.pallas_call(
        body,
        out_shape=jax.ShapeDtypeStruct((ROWS, D), jnp.float32),
        in_specs=[
            pl.BlockSpec(memory_space=pltpu.VMEM),
            pl.BlockSpec(memory_space=pltpu.VMEM),
            pl.BlockSpec(memory_space=pltpu.VMEM),
            pl.BlockSpec(memory_space=pl.ANY),
            pl.BlockSpec(memory_space=pl.ANY),
        ],
        out_specs=pl.BlockSpec(memory_space=pltpu.VMEM),
        scratch_shapes=[
            pltpu.VMEM((2, H, Skv, Dh), jnp.float32),
            pltpu.VMEM((2, H, Skv, Dh), jnp.float32),
            pltpu.VMEM((N_DEV - 1, CHUNK, D), jnp.float32),
            pltpu.VMEM((Sq, H * Dh), jnp.bfloat16),
            pltpu.VMEM((Sq, H * Dh), jnp.bfloat16),
            pltpu.SemaphoreType.DMA((4,)),
            pltpu.SemaphoreType.DMA((N_DEV - 1,)),
            pltpu.SemaphoreType.DMA((N_DEV - 1,)),
            pltpu.SemaphoreType.DMA((3,)),
            pltpu.SemaphoreType.DMA((3,)),
        ],
        compiler_params=pltpu.CompilerParams(collective_id=0),
    )(xm, Wq, Wo, K_ext, V_ext)
    return out2.reshape(B, Sq, D)


# device time: 55755 ns/iter; 1.7423x vs baseline; 1.5407x over previous
import jax
import jax.numpy as jnp
from jax import lax
from jax.experimental import pallas as pl
from jax.experimental.pallas import tpu as pltpu

N_DEV = 4
B, Sq, Skv, Dh = 4, 256, 1024, 128
H = 8
HG = 32
D = 1024
SCALE = 0.08838834764831843
ROWS = B * Sq
CHUNK = ROWS // N_DEV


def kernel(x, Wq, Wo, K_ext, V_ext):
    xm = x.reshape(ROWS, D)

    def body(x_ref, wq_ref, wo_ref, k_hbm, v_hbm, out_ref,
             kbuf, vbuf, comm_ref, q_ref, attn_ref, stage_ref, bcbuf,
             copy_sems, rs_send_sems, rs_recv_sems, bc_send_sems,
             bc_recv_sems):
        my = lax.axis_index("i")
        left = (my + N_DEV - 1) % N_DEV
        right = (my + 1) % N_DEV
        diag = (my + 2) % N_DEV
        hs = my * H

        def kv_copy(b, slot):
            copies = []
            for h in range(H):
                copies.append(pltpu.make_async_copy(
                    k_hbm.at[b, :, hs + h, :], kbuf.at[slot, h],
                    copy_sems.at[2 * slot]))
                copies.append(pltpu.make_async_copy(
                    v_hbm.at[b, :, hs + h, :], vbuf.at[slot, h],
                    copy_sems.at[2 * slot + 1]))
            for c in copies:
                c.start()
            return copies

        def batch_index(j):
            if j < N_DEV - 1:
                return (my + N_DEV - j) % N_DEV
            return (my + 1) % N_DEV

        pending = kv_copy(batch_index(0), 0)

        barrier_sem = pltpu.get_barrier_semaphore()
        for nbr in (left, right):
            pl.semaphore_signal(
                barrier_sem, inc=1,
                device_id=(nbr,), device_id_type=pl.DeviceIdType.MESH,
            )
        pl.semaphore_wait(barrier_sem, 2)

        bf16 = jnp.bfloat16
        wqb = wq_ref[...].astype(bf16)
        wob = wo_ref[...].astype(bf16)

        rdmas = []
        for j in range(N_DEV):
            b = batch_index(j)
            slot = j % 2
            q_ref[...] = (jnp.dot(
                x_ref[pl.ds(b * Sq, Sq), :].astype(bf16), wqb,
                preferred_element_type=jnp.float32)
                * SCALE).astype(bf16)
            for c in pending:
                c.wait()
            if j + 1 < N_DEV:
                pending = kv_copy(batch_index(j + 1), (j + 1) % 2)

            def head_step(h, _, slot=slot):
                qh = q_ref[:, pl.ds(h * Dh, Dh)]
                kh = kbuf[slot, h].astype(bf16)
                vh = vbuf[slot, h].astype(bf16)
                s = lax.dot_general(
                    qh, kh, (((1,), (1,)), ((), ())),
                    preferred_element_type=jnp.float32)
                p = jnp.exp(s)
                l = jnp.sum(p, axis=-1, keepdims=True)
                o = jnp.dot(p.astype(bf16), vh,
                            preferred_element_type=jnp.float32) / l
                attn_ref[:, pl.ds(h * Dh, Dh)] = o.astype(bf16)
                return 0

            lax.fori_loop(0, H, head_step, 0, unroll=4)
            part = jnp.dot(attn_ref[...], wob,
                           preferred_element_type=jnp.float32)

            if j == 0:
                chunk = part
            else:
                prev = rdmas[j - 1]
                prev.wait_recv()
                chunk = part + comm_ref[j - 1].astype(jnp.float32)
            out_ref[pl.ds(b * CHUNK, CHUNK), :] = chunk
            if j < N_DEV - 1:
                stage_ref[j] = chunk.astype(bf16)
                rdma = pltpu.make_async_remote_copy(
                    src_ref=stage_ref.at[j],
                    dst_ref=comm_ref.at[j],
                    send_sem=rs_send_sems.at[j],
                    recv_sem=rs_recv_sems.at[j],
                    device_id=(right,),
                    device_id_type=pl.DeviceIdType.MESH,
                )
                rdma.start()
                rdmas.append(rdma)
            else:
                stage_ref[N_DEV - 1] = chunk.astype(bf16)

        for k, tgt in enumerate((right, left, diag)):
            bc = pltpu.make_async_remote_copy(
                src_ref=stage_ref.at[N_DEV - 1],
                dst_ref=bcbuf.at[k],
                send_sem=bc_send_sems.at[k],
                recv_sem=bc_recv_sems.at[k],
                device_id=(tgt,),
                device_id_type=pl.DeviceIdType.MESH,
            )
            bc.start()
            rdmas.append(bc)
        for k in range(3):
            rdmas[N_DEV - 1 + k].wait_recv()
            c = (my + (0, 2, 3)[k]) % N_DEV
            out_ref[pl.ds(c * CHUNK, CHUNK), :] = bcbuf[k].astype(jnp.float32)
        for r in rdmas:
            r.wait_send()

    out2 = pl.pallas_call(
        body,
        out_shape=jax.ShapeDtypeStruct((ROWS, D), jnp.float32),
        in_specs=[
            pl.BlockSpec(memory_space=pltpu.VMEM),
            pl.BlockSpec(memory_space=pltpu.VMEM),
            pl.BlockSpec(memory_space=pltpu.VMEM),
            pl.BlockSpec(memory_space=pl.ANY),
            pl.BlockSpec(memory_space=pl.ANY),
        ],
        out_specs=pl.BlockSpec(memory_space=pltpu.VMEM),
        scratch_shapes=[
            pltpu.VMEM((2, H, Skv, Dh), jnp.float32),
            pltpu.VMEM((2, H, Skv, Dh), jnp.float32),
            pltpu.VMEM((N_DEV - 1, CHUNK, D), jnp.bfloat16),
            pltpu.VMEM((Sq, H * Dh), jnp.bfloat16),
            pltpu.VMEM((Sq, H * Dh), jnp.bfloat16),
            pltpu.VMEM((N_DEV, CHUNK, D), jnp.bfloat16),
            pltpu.VMEM((3, CHUNK, D), jnp.bfloat16),
            pltpu.SemaphoreType.DMA((4,)),
            pltpu.SemaphoreType.DMA((N_DEV - 1,)),
            pltpu.SemaphoreType.DMA((N_DEV - 1,)),
            pltpu.SemaphoreType.DMA((3,)),
            pltpu.SemaphoreType.DMA((3,)),
        ],
        compiler_params=pltpu.CompilerParams(collective_id=0),
    )(xm, Wq, Wo, K_ext, V_ext)
    return out2.reshape(B, Sq, D)


# device time: 55323 ns/iter; 1.7559x vs baseline; 1.0078x over previous
import jax
import jax.numpy as jnp
from jax import lax
from jax.experimental import pallas as pl
from jax.experimental.pallas import tpu as pltpu

N_DEV = 4
B, Sq, Skv, Dh = 4, 256, 1024, 128
H = 8
HG = 32
D = 1024
SCALE = 0.08838834764831843
SCALE_LOG2E = SCALE * 1.4426950408889634
ROWS = B * Sq
CHUNK = ROWS // N_DEV


def kernel(x, Wq, Wo, K_ext, V_ext):
    xm = x.reshape(ROWS, D)

    def body(x_ref, wq_ref, wo_ref, k_hbm, v_hbm, out_ref,
             kbuf, vbuf, comm_ref, q_ref, attn_ref, stage_ref, bcbuf,
             copy_sems, rs_send_sems, rs_recv_sems, bc_send_sems,
             bc_recv_sems):
        my = lax.axis_index("i")
        left = (my + N_DEV - 1) % N_DEV
        right = (my + 1) % N_DEV
        diag = (my + 2) % N_DEV
        hs = my * H

        def kv_copy(b, slot):
            copies = []
            for h in range(H):
                copies.append(pltpu.make_async_copy(
                    k_hbm.at[b, :, hs + h, :], kbuf.at[slot, h],
                    copy_sems.at[2 * slot]))
                copies.append(pltpu.make_async_copy(
                    v_hbm.at[b, :, hs + h, :], vbuf.at[slot, h],
                    copy_sems.at[2 * slot + 1]))
            for c in copies:
                c.start()
            return copies

        def batch_index(j):
            if j < N_DEV - 1:
                return (my + N_DEV - j) % N_DEV
            return (my + 1) % N_DEV

        pending = kv_copy(batch_index(0), 0)

        barrier_sem = pltpu.get_barrier_semaphore()
        for nbr in (left, right):
            pl.semaphore_signal(
                barrier_sem, inc=1,
                device_id=(nbr,), device_id_type=pl.DeviceIdType.MESH,
            )
        pl.semaphore_wait(barrier_sem, 2)

        bf16 = jnp.bfloat16
        wqb = wq_ref[...].astype(bf16)
        wob = wo_ref[...].astype(bf16)

        rdmas = []
        for j in range(N_DEV):
            b = batch_index(j)
            slot = j % 2
            q_ref[...] = (jnp.dot(
                x_ref[pl.ds(b * Sq, Sq), :].astype(bf16), wqb,
                preferred_element_type=jnp.float32)
                * SCALE_LOG2E).astype(bf16)
            for c in pending:
                c.wait()
            if j + 1 < N_DEV:
                pending = kv_copy(batch_index(j + 1), (j + 1) % 2)

            def head_step(h, _, slot=slot):
                qh = q_ref[:, pl.ds(h * Dh, Dh)]
                kh = kbuf[slot, h].astype(bf16)
                vh = vbuf[slot, h].astype(bf16)
                s = lax.dot_general(
                    qh, kh, (((1,), (1,)), ((), ())),
                    preferred_element_type=jnp.float32)
                p = jnp.exp2(s)
                l = jnp.sum(p, axis=-1, keepdims=True)
                o = jnp.dot(p.astype(bf16), vh,
                            preferred_element_type=jnp.float32) / l
                attn_ref[:, pl.ds(h * Dh, Dh)] = o.astype(bf16)
                return 0

            lax.fori_loop(0, H, head_step, 0, unroll=8)
            part = jnp.dot(attn_ref[...], wob,
                           preferred_element_type=jnp.float32)

            if j == 0:
                chunk = part
            else:
                prev = rdmas[j - 1]
                prev.wait_recv()
                chunk = part + comm_ref[j - 1].astype(jnp.float32)
            out_ref[pl.ds(b * CHUNK, CHUNK), :] = chunk
            if j < N_DEV - 1:
                stage_ref[j] = chunk.astype(bf16)
                rdma = pltpu.make_async_remote_copy(
                    src_ref=stage_ref.at[j],
                    dst_ref=comm_ref.at[j],
                    send_sem=rs_send_sems.at[j],
                    recv_sem=rs_recv_sems.at[j],
                    device_id=(right,),
                    device_id_type=pl.DeviceIdType.MESH,
                )
                rdma.start()
                rdmas.append(rdma)
            else:
                stage_ref[N_DEV - 1] = chunk.astype(bf16)

        for k, tgt in enumerate((right, left, diag)):
            bc = pltpu.make_async_remote_copy(
                src_ref=stage_ref.at[N_DEV - 1],
                dst_ref=bcbuf.at[k],
                send_sem=bc_send_sems.at[k],
                recv_sem=bc_recv_sems.at[k],
                device_id=(tgt,),
                device_id_type=pl.DeviceIdType.MESH,
            )
            bc.start()
            rdmas.append(bc)
        for k in range(3):
            rdmas[N_DEV - 1 + k].wait_recv()
            c = (my + (0, 2, 3)[k]) % N_DEV
            out_ref[pl.ds(c * CHUNK, CHUNK), :] = bcbuf[k].astype(jnp.float32)
        for r in rdmas:
            r.wait_send()

    out2 = pl.pallas_call(
        body,
        out_shape=jax.ShapeDtypeStruct((ROWS, D), jnp.float32),
        in_specs=[
            pl.BlockSpec(memory_space=pltpu.VMEM),
            pl.BlockSpec(memory_space=pltpu.VMEM),
            pl.BlockSpec(memory_space=pltpu.VMEM),
            pl.BlockSpec(memory_space=pl.ANY),
            pl.BlockSpec(memory_space=pl.ANY),
        ],
        out_specs=pl.BlockSpec(memory_space=pltpu.VMEM),
        scratch_shapes=[
            pltpu.VMEM((2, H, Skv, Dh), jnp.float32),
            pltpu.VMEM((2, H, Skv, Dh), jnp.float32),
            pltpu.VMEM((N_DEV - 1, CHUNK, D), jnp.bfloat16),
            pltpu.VMEM((Sq, H * Dh), jnp.bfloat16),
            pltpu.VMEM((Sq, H * Dh), jnp.bfloat16),
            pltpu.VMEM((N_DEV, CHUNK, D), jnp.bfloat16),
            pltpu.VMEM((3, CHUNK, D), jnp.bfloat16),
            pltpu.SemaphoreType.DMA((4,)),
            pltpu.SemaphoreType.DMA((N_DEV - 1,)),
            pltpu.SemaphoreType.DMA((N_DEV - 1,)),
            pltpu.SemaphoreType.DMA((3,)),
            pltpu.SemaphoreType.DMA((3,)),
        ],
        compiler_params=pltpu.CompilerParams(collective_id=0),
    )(xm, Wq, Wo, K_ext, V_ext)
    return out2.reshape(B, Sq, D)
